# baseline (speedup 1.0000x reference)
import jax
import jax.numpy as jnp
from jax.experimental import pallas as pl
from jax.experimental.pallas import tpu as pltpu

_ALPHA = 0.3
_LANE = 128


def _gat_kernel(x_ref, w_ref, b_ref, al_ref, ar_ref, out_ref, *, num_heads, c_head):
    x = x_ref[0]
    nrm = jnp.sqrt(jnp.sum(x * x, axis=1, keepdims=True))
    n = x / jnp.maximum(nrm, 1e-12)
    sim = jax.lax.dot_general(n, n, (((1,), (1,)), ((), ())),
                              preferred_element_type=jnp.float32)
    mask_f = jnp.where(sim > 0.0, 1.0, 0.0)
    fe = jax.lax.dot_general(x, w_ref[...], (((1,), (1,)), ((), ())),
                             preferred_element_type=jnp.float32)
    fe = fe + b_ref[...][None, :]
    lcol = jnp.dot(fe, al_ref[...],
                   preferred_element_type=jnp.float32)
    rcol = jnp.dot(fe, ar_ref[...],
                   preferred_element_type=jnp.float32)
    rrow = rcol.T
    for h in range(num_heads):
        l_h = lcol[:, h:h + 1]
        r_h = rcol[:, h:h + 1]
        ge = rrow[h:h + 1, :] >= (0.0 - l_h)
        A = jnp.where(ge, mask_f, 0.0)
        Bm = mask_f - A
        mr = jnp.max(r_h)
        er = jnp.exp(r_h - mr)
        ear = jnp.exp(_ALPHA * (r_h - mr))
        fe_h = fe[:, h * _LANE:(h + 1) * _LANE]
        AG = jnp.dot(A, er * fe_h, preferred_element_type=jnp.float32)
        BG = jnp.dot(Bm, ear * fe_h, preferred_element_type=jnp.float32)
        t1 = l_h + mr
        t2 = _ALPHA * t1
        c = jnp.maximum(t1, t2)
        w1 = jnp.exp(t1 - c)
        w2 = jnp.exp(t2 - c)
        num = w1 * AG[:, :c_head] + w2 * BG[:, :c_head]
        s = w1 * AG[:, c_head:c_head + 1] + w2 * BG[:, c_head:c_head + 1]
        out_ref[0, :, h * c_head:(h + 1) * c_head] = num / s


def kernel(node_feats, W, b, a):
    B, N, C = node_feats.shape
    H = a.shape[0]
    c_head = a.shape[1] // 2
    O = H * c_head
    P = H * _LANE
    Wx = jnp.zeros((P, C), W.dtype)
    bx = jnp.zeros((P,), b.dtype)
    Alx = jnp.zeros((P, H), a.dtype)
    Arx = jnp.zeros((P, H), a.dtype)
    for h in range(H):
        Wx = Wx.at[h * _LANE:h * _LANE + c_head].set(
            W[h * c_head:(h + 1) * c_head])
        bx = bx.at[h * _LANE:h * _LANE + c_head].set(
            b[h * c_head:(h + 1) * c_head])
        bx = bx.at[h * _LANE + c_head].set(1.0)
        Alx = Alx.at[h * _LANE:h * _LANE + c_head, h].set(a[h, :c_head])
        Arx = Arx.at[h * _LANE:h * _LANE + c_head, h].set(a[h, c_head:])

    grid = (B,)
    out = pl.pallas_call(
        lambda *refs: _gat_kernel(*refs, num_heads=H, c_head=c_head),
        grid=grid,
        in_specs=[
            pl.BlockSpec((1, N, C), lambda i: (i, 0, 0)),
            pl.BlockSpec((P, C), lambda i: (0, 0)),
            pl.BlockSpec((P,), lambda i: (0,)),
            pl.BlockSpec((P, H), lambda i: (0, 0)),
            pl.BlockSpec((P, H), lambda i: (0, 0)),
        ],
        out_specs=pl.BlockSpec((1, N, O), lambda i: (i, 0, 0)),
        out_shape=jax.ShapeDtypeStruct((B, N, O), jnp.float32),
        compiler_params=pltpu.CompilerParams(
            dimension_semantics=("parallel",)),
    )(node_feats, Wx, bx, Alx, Arx)
    return out


# speedup vs baseline: 1.1745x; 1.1745x over previous
import jax
import jax.numpy as jnp
from jax.experimental import pallas as pl
from jax.experimental.pallas import tpu as pltpu

_ALPHA = 0.3
_LANE = 128


def _gat_kernel(x_ref, w_ref, b_ref, al_ref, ar_ref, out_ref, *, num_heads, c_head):
    x = x_ref[0]
    nrm = jnp.sqrt(jnp.sum(x * x, axis=1, keepdims=True))
    n = x / jnp.maximum(nrm, 1e-12)
    sim = jax.lax.dot_general(n, n, (((1,), (1,)), ((), ())),
                              preferred_element_type=jnp.float32)
    mask_f = jnp.where(sim > 0.0, 1.0, 0.0)
    fe = jax.lax.dot_general(x, w_ref[...], (((1,), (1,)), ((), ())),
                             preferred_element_type=jnp.float32)
    fe = fe + b_ref[...][None, :]
    lcol = jnp.dot(fe, al_ref[...],
                   preferred_element_type=jnp.float32)
    rcol = jnp.dot(fe, ar_ref[...],
                   preferred_element_type=jnp.float32)
    rrow = rcol.T
    for h in range(num_heads):
        l_h = lcol[:, h:h + 1]
        r_h = rcol[:, h:h + 1]
        ge = rrow[h:h + 1, :] >= (0.0 - l_h)
        A = jnp.where(ge, mask_f, 0.0)
        Bm = mask_f - A
        mr = jnp.max(r_h)
        er = jnp.exp(r_h - mr)
        ear = jnp.exp(_ALPHA * (r_h - mr))
        fe_h = fe[:, h * _LANE:(h + 1) * _LANE]
        AG = jnp.dot(A, er * fe_h, preferred_element_type=jnp.float32)
        BG = jnp.dot(Bm, ear * fe_h, preferred_element_type=jnp.float32)
        t1 = l_h + mr
        t2 = _ALPHA * t1
        c = jnp.maximum(t1, t2)
        w1 = jnp.exp(t1 - c)
        w2 = jnp.exp(t2 - c)
        num = w1 * AG[:, :c_head] + w2 * BG[:, :c_head]
        s = w1 * AG[:, c_head:c_head + 1] + w2 * BG[:, c_head:c_head + 1]
        out_ref[0, :, h * c_head:(h + 1) * c_head] = num / s


def kernel(node_feats, W, b, a):
    B, N, C = node_feats.shape
    H = a.shape[0]
    c_head = a.shape[1] // 2
    O = H * c_head
    P = H * _LANE
    pad = _LANE - c_head
    Wx = jnp.pad(W.reshape(H, c_head, C), ((0, 0), (0, pad), (0, 0))).reshape(P, C)
    ones_col = (jnp.arange(P) % _LANE == c_head).astype(b.dtype)
    bx = jnp.pad(b.reshape(H, c_head), ((0, 0), (0, pad))).reshape(P) + ones_col
    eye = jnp.eye(H, dtype=a.dtype)
    alp = jnp.pad(a[:, :c_head], ((0, 0), (0, pad)))
    arp = jnp.pad(a[:, c_head:], ((0, 0), (0, pad)))
    Alx = (alp[:, :, None] * eye[:, None, :]).reshape(P, H)
    Arx = (arp[:, :, None] * eye[:, None, :]).reshape(P, H)

    grid = (B,)
    out = pl.pallas_call(
        lambda *refs: _gat_kernel(*refs, num_heads=H, c_head=c_head),
        grid=grid,
        in_specs=[
            pl.BlockSpec((1, N, C), lambda i: (i, 0, 0)),
            pl.BlockSpec((P, C), lambda i: (0, 0)),
            pl.BlockSpec((P,), lambda i: (0,)),
            pl.BlockSpec((P, H), lambda i: (0, 0)),
            pl.BlockSpec((P, H), lambda i: (0, 0)),
        ],
        out_specs=pl.BlockSpec((1, N, O), lambda i: (i, 0, 0)),
        out_shape=jax.ShapeDtypeStruct((B, N, O), jnp.float32),
        compiler_params=pltpu.CompilerParams(
            dimension_semantics=("parallel",)),
    )(node_feats, Wx, bx, Alx, Arx)
    return out


# speedup vs baseline: 1.3405x; 1.1413x over previous
import jax
import jax.numpy as jnp
from jax.experimental import pallas as pl
from jax.experimental.pallas import tpu as pltpu

_ALPHA = 0.3


def _gat_kernel(x_ref, w_ref, b_ref, al_ref, ar_ref, out_ref, g1_ref, g2_ref,
                *, num_heads, c_head):
    x = x_ref[0]
    N = x.shape[0]
    nrm = jnp.sqrt(jnp.sum(x * x, axis=1, keepdims=True))
    n = x / jnp.maximum(nrm, 1e-12)
    sim = jax.lax.dot_general(n, n, (((1,), (1,)), ((), ())),
                              preferred_element_type=jnp.float32)
    mask_f = jnp.where(sim > 0.0, 1.0, 0.0)
    feats = jax.lax.dot_general(x, w_ref[...], (((1,), (1,)), ((), ())),
                                preferred_element_type=jnp.float32)
    feats = feats + b_ref[...][None, :]
    lcol = jnp.dot(feats, al_ref[...],
                   preferred_element_type=jnp.float32)
    rcol = jnp.dot(feats, ar_ref[...],
                   preferred_element_type=jnp.float32)
    rrow = rcol.T
    for h in range(num_heads):
        l_h = lcol[:, h:h + 1]
        r_h = rcol[:, h:h + 1]
        ge = rrow[h:h + 1, :] >= (0.0 - l_h)
        A = jnp.where(ge, mask_f, 0.0)
        Bm = mask_f - A
        mr = jnp.max(r_h)
        er = jnp.exp(r_h - mr)
        ear = jnp.exp(_ALPHA * (r_h - mr))
        f_h = feats[:, h * c_head:(h + 1) * c_head]
        g1_ref[:, :c_head] = er * f_h
        g1_ref[:, c_head:] = jnp.broadcast_to(er, (N, c_head))
        g2_ref[:, :c_head] = ear * f_h
        g2_ref[:, c_head:] = jnp.broadcast_to(ear, (N, c_head))
        AG = jnp.dot(A, g1_ref[...], preferred_element_type=jnp.float32)
        BG = jnp.dot(Bm, g2_ref[...], preferred_element_type=jnp.float32)
        t1 = l_h + mr
        t2 = _ALPHA * t1
        c = jnp.maximum(t1, t2)
        w1 = jnp.exp(t1 - c)
        w2 = jnp.exp(t2 - c)
        num = w1 * AG[:, :c_head] + w2 * BG[:, :c_head]
        s = w1 * AG[:, c_head:c_head + 1] + w2 * BG[:, c_head:c_head + 1]
        out_ref[0, :, h * c_head:(h + 1) * c_head] = num / s


def kernel(node_feats, W, b, a):
    B, N, C = node_feats.shape
    H = a.shape[0]
    c_head = a.shape[1] // 2
    O = H * c_head
    eye = jnp.eye(H, dtype=a.dtype)
    Al = (a[:, :c_head, None] * eye[:, None, :]).reshape(O, H)
    Ar = (a[:, c_head:, None] * eye[:, None, :]).reshape(O, H)

    grid = (B,)
    out = pl.pallas_call(
        lambda *refs: _gat_kernel(*refs, num_heads=H, c_head=c_head),
        grid=grid,
        in_specs=[
            pl.BlockSpec((1, N, C), lambda i: (i, 0, 0)),
            pl.BlockSpec((O, C), lambda i: (0, 0)),
            pl.BlockSpec((O,), lambda i: (0,)),
            pl.BlockSpec((O, H), lambda i: (0, 0)),
            pl.BlockSpec((O, H), lambda i: (0, 0)),
        ],
        out_specs=pl.BlockSpec((1, N, O), lambda i: (i, 0, 0)),
        out_shape=jax.ShapeDtypeStruct((B, N, O), jnp.float32),
        scratch_shapes=[
            pltpu.VMEM((N, 2 * c_head), jnp.float32),
            pltpu.VMEM((N, 2 * c_head), jnp.float32),
        ],
        compiler_params=pltpu.CompilerParams(
            dimension_semantics=("parallel",)),
    )(node_feats, W, b, Al, Ar)
    return out
